# baseline (speedup 1.0000x reference)
import functools

import jax
import jax.numpy as jnp
from jax import lax
from jax.experimental import pallas as pl
from jax.experimental.pallas import tpu as pltpu
from jax.experimental.pallas import tpu_sc as plsc

N_NODES = 10000
N_EDGES = 160000
D_IN = 256
D_OUT = 256
D_HALF = D_IN // 2

NC = 2
NS = 16
L = 16
CHUNK = 16
N_CHUNKS = 640
E_PER_TILE = N_CHUNKS * CHUNK
E_PAD = E_PER_TILE * NS
K_RING = 16
N_ROUNDS = N_CHUNKS // K_RING
N_ACC = 10112
ROWS_PER_TILE_Z = N_ACC // NS
ROWS_PER_TILE_O = 624
TAIL_O = N_NODES - ROWS_PER_TILE_O * NS


def _sc_segment_gather_sum(packed, x2):
    mesh = plsc.VectorSubcoreMesh(core_axis_name="c", subcore_axis_name="s")

    @functools.partial(
        pl.kernel,
        mesh=mesh,
        out_type=jax.ShapeDtypeStruct((2 * N_NODES, D_HALF), jnp.float32),
        scratch_types=[
            pltpu.VMEM((N_CHUNKS // 8, 8 * CHUNK), jnp.int32),
        ]
        + [pltpu.VMEM((CHUNK,), jnp.int32)] * (2 * K_RING)
        + [pltpu.VMEM((CHUNK, D_HALF), jnp.float32)] * K_RING
        + [pltpu.VMEM_SHARED((N_ACC, D_HALF), jnp.float32)]
        + [pltpu.SemaphoreType.DMA] * K_RING,
    )
    def k(packed_hbm, x2_hbm, out_hbm, pk_v, *bufs):
        c = lax.axis_index("c")
        s = lax.axis_index("s")
        gbufs = [bufs[2 * t] for t in range(K_RING)]
        dbufs = [bufs[2 * t + 1] for t in range(K_RING)]
        rows = list(bufs[2 * K_RING:3 * K_RING])
        h_sh = bufs[3 * K_RING]
        sems = list(bufs[3 * K_RING + 1:])
        r0 = rows[0]

        pltpu.sync_copy(packed_hbm.at[c, s], pk_v)

        def unpack(j8, eighth, b):
            for kk in range(CHUNK // L):
                v = pk_v[j8, pl.ds(eighth * CHUNK + kk * L, L)]
                gbufs[b][pl.ds(kk * L, L)] = v & 0xFFFF
                dbufs[b][pl.ds(kk * L, L)] = lax.shift_right_logical(v, 16)

        def gather(b):
            return pltpu.make_async_copy(x2_hbm.at[gbufs[b]], rows[b], sems[b])

        zv = jnp.zeros((L,), jnp.float32)

        def zero_body(r, carry):
            for kk in range(D_HALF // L):
                r0[r, pl.ds(kk * L, L)] = zv
            return carry

        lax.fori_loop(0, CHUNK, zero_body, 0)
        zbase = s * ROWS_PER_TILE_Z
        for zk in range(ROWS_PER_TILE_Z // CHUNK):
            pltpu.sync_copy(r0, h_sh.at[pl.ds(zbase + zk * CHUNK, CHUNK)])
        ztail = ROWS_PER_TILE_Z % CHUNK
        pltpu.sync_copy(
            r0.at[pl.ds(0, ztail)],
            h_sh.at[pl.ds(zbase + (ROWS_PER_TILE_Z // CHUNK) * CHUNK, ztail)])

        for b in range(K_RING):
            unpack(b // 8, b % 8, b)
            gather(b).start()
        plsc.subcore_barrier()

        def round_body(i, carry):
            for b in range(K_RING):
                gather(b).wait()
                pltpu.sync_copy(rows[b], h_sh.at[dbufs[b]], add=True)
                unpack(2 * i + 2 + b // 8, b % 8, b)
                gather(b).start()
            return carry

        lax.fori_loop(0, N_ROUNDS - 1, round_body, 0)
        for b in range(K_RING):
            gather(b).wait()
            pltpu.sync_copy(rows[b], h_sh.at[dbufs[b]], add=True)
        plsc.subcore_barrier()

        pltpu.sync_copy(
            h_sh.at[pl.ds(s * ROWS_PER_TILE_O, ROWS_PER_TILE_O)],
            out_hbm.at[pl.ds(c * N_NODES + s * ROWS_PER_TILE_O, ROWS_PER_TILE_O)])

        @pl.when(s == 0)
        def _tail():
            pltpu.sync_copy(
                h_sh.at[pl.ds(NS * ROWS_PER_TILE_O, TAIL_O)],
                out_hbm.at[pl.ds(c * N_NODES + NS * ROWS_PER_TILE_O, TAIL_O)])

    return k(packed, x2)


def _mm_body(h0_ref, h1_ref, wt0_ref, wt1_ref, b_ref, o_ref):
    o_ref[...] = (
        jnp.dot(h0_ref[...], wt0_ref[...], preferred_element_type=jnp.float32)
        + jnp.dot(h1_ref[...], wt1_ref[...], preferred_element_type=jnp.float32)
        + b_ref[...]
    )


def _linear(h2, Wt, b):
    R = 1000
    grid = (N_NODES // R,)
    return pl.pallas_call(
        _mm_body,
        grid=grid,
        in_specs=[
            pl.BlockSpec((R, D_HALF), lambda i: (i, 0)),
            pl.BlockSpec((R, D_HALF), lambda i: (i + N_NODES // R, 0)),
            pl.BlockSpec((D_HALF, D_OUT), lambda i: (0, 0)),
            pl.BlockSpec((D_HALF, D_OUT), lambda i: (1, 0)),
            pl.BlockSpec((1, D_OUT), lambda i: (0, 0)),
        ],
        out_specs=pl.BlockSpec((R, D_OUT), lambda i: (i, 0)),
        out_shape=jax.ShapeDtypeStruct((N_NODES, D_OUT), jnp.float32),
    )(h2, h2, Wt, Wt, b.reshape(1, D_OUT))


def kernel(x, edge_index, W, b):
    src = edge_index[0].astype(jnp.int32)
    dst = edge_index[1].astype(jnp.int32)
    pad = E_PAD - N_EDGES
    x2 = x.reshape(N_NODES, 2, D_HALF).reshape(2 * N_NODES, D_HALF)
    g0 = jnp.concatenate([src * 2, jnp.zeros((pad,), jnp.int32)])
    dstp = jnp.concatenate([dst, jnp.full((pad,), N_NODES, jnp.int32)])
    base = dstp * 65536 + g0
    packed = jnp.stack([base, base + 1]).reshape(2, NS, N_CHUNKS // 8, 8 * CHUNK)
    h2 = _sc_segment_gather_sum(packed, x2)
    return _linear(h2, W.T, b)


# speedup vs baseline: 1.0301x; 1.0301x over previous
import functools

import jax
import jax.numpy as jnp
from jax import lax
from jax.experimental import pallas as pl
from jax.experimental.pallas import tpu as pltpu
from jax.experimental.pallas import tpu_sc as plsc

N_NODES = 10000
N_EDGES = 160000
D_IN = 256
D_OUT = 256
D_HALF = D_IN // 2

NC = 2
NS = 16
L = 16
CHUNK = 32
N_CHUNKS = 320
E_PER_TILE = N_CHUNKS * CHUNK
E_PAD = E_PER_TILE * NS
K_RING = 8
N_ROUNDS = N_CHUNKS // K_RING
N_ACC = 10112
ROWS_PER_TILE_Z = N_ACC // NS
ROWS_PER_TILE_O = 624
TAIL_O = N_NODES - ROWS_PER_TILE_O * NS


def _sc_segment_gather_sum(packed, x2):
    mesh = plsc.VectorSubcoreMesh(core_axis_name="c", subcore_axis_name="s")

    @functools.partial(
        pl.kernel,
        mesh=mesh,
        out_type=jax.ShapeDtypeStruct((2 * N_NODES, D_HALF), jnp.float32),
        scratch_types=[
            pltpu.VMEM((N_CHUNKS // 4, 4 * CHUNK), jnp.int32),
        ]
        + [pltpu.VMEM((CHUNK,), jnp.int32)] * (2 * K_RING)
        + [pltpu.VMEM((CHUNK, D_HALF), jnp.float32)] * K_RING
        + [pltpu.VMEM_SHARED((N_ACC, D_HALF), jnp.float32)]
        + [pltpu.SemaphoreType.DMA] * K_RING,
    )
    def k(packed_hbm, x2_hbm, out_hbm,
          pk_v, g0b, d0b, g1b, d1b, g2b, d2b, g3b, d3b, g4b, d4b, g5b, d5b,
          g6b, d6b, g7b, d7b, r0, r1, r2, r3, r4, r5, r6, r7, h_sh,
          s0, s1, s2, s3, s4, s5, s6, s7):
        c = lax.axis_index("c")
        s = lax.axis_index("s")
        gbufs = [g0b, g1b, g2b, g3b, g4b, g5b, g6b, g7b]
        dbufs = [d0b, d1b, d2b, d3b, d4b, d5b, d6b, d7b]
        rows = [r0, r1, r2, r3, r4, r5, r6, r7]
        sems = [s0, s1, s2, s3, s4, s5, s6, s7]

        pltpu.sync_copy(packed_hbm.at[c, s], pk_v)

        def unpack(j4, quarter, b):
            for kk in range(CHUNK // L):
                v = pk_v[j4, pl.ds(quarter * CHUNK + kk * L, L)]
                gbufs[b][pl.ds(kk * L, L)] = v & 0xFFFF
                dbufs[b][pl.ds(kk * L, L)] = lax.shift_right_logical(v, 16)

        def gather(b):
            return pltpu.make_async_copy(x2_hbm.at[gbufs[b]], rows[b], sems[b])

        zv = jnp.zeros((L,), jnp.float32)

        def zero_body(r, carry):
            for kk in range(D_HALF // L):
                r0[r, pl.ds(kk * L, L)] = zv
            return carry

        lax.fori_loop(0, CHUNK, zero_body, 0)
        zbase = s * ROWS_PER_TILE_Z
        for zk in range(ROWS_PER_TILE_Z // CHUNK):
            pltpu.sync_copy(r0, h_sh.at[pl.ds(zbase + zk * CHUNK, CHUNK)])
        ztail = ROWS_PER_TILE_Z % CHUNK
        pltpu.sync_copy(
            r0.at[pl.ds(0, ztail)],
            h_sh.at[pl.ds(zbase + (ROWS_PER_TILE_Z // CHUNK) * CHUNK, ztail)])

        for b in range(K_RING):
            unpack(b // 4, b % 4, b)
            gather(b).start()
        plsc.subcore_barrier()

        def round_body(i, carry):
            for b in range(K_RING):
                gather(b).wait()
                pltpu.sync_copy(rows[b], h_sh.at[dbufs[b]], add=True)
                unpack(2 * i + 2 + b // 4, b % 4, b)
                gather(b).start()
            return carry

        lax.fori_loop(0, N_ROUNDS - 1, round_body, 0)
        for b in range(K_RING):
            gather(b).wait()
            pltpu.sync_copy(rows[b], h_sh.at[dbufs[b]], add=True)
        plsc.subcore_barrier()

        pltpu.sync_copy(
            h_sh.at[pl.ds(s * ROWS_PER_TILE_O, ROWS_PER_TILE_O)],
            out_hbm.at[pl.ds(c * N_NODES + s * ROWS_PER_TILE_O, ROWS_PER_TILE_O)])

        @pl.when(s == 0)
        def _tail():
            pltpu.sync_copy(
                h_sh.at[pl.ds(NS * ROWS_PER_TILE_O, TAIL_O)],
                out_hbm.at[pl.ds(c * N_NODES + NS * ROWS_PER_TILE_O, TAIL_O)])

    return k(packed, x2)


def _mm_body(h0_ref, h1_ref, wt0_ref, wt1_ref, b_ref, o_ref):
    o_ref[...] = (
        jnp.dot(h0_ref[...], wt0_ref[...], preferred_element_type=jnp.float32)
        + jnp.dot(h1_ref[...], wt1_ref[...], preferred_element_type=jnp.float32)
        + b_ref[...]
    )


def _linear(h2, Wt, b):
    R = 1000
    grid = (N_NODES // R,)
    return pl.pallas_call(
        _mm_body,
        grid=grid,
        in_specs=[
            pl.BlockSpec((R, D_HALF), lambda i: (i, 0)),
            pl.BlockSpec((R, D_HALF), lambda i: (i + N_NODES // R, 0)),
            pl.BlockSpec((D_HALF, D_OUT), lambda i: (0, 0)),
            pl.BlockSpec((D_HALF, D_OUT), lambda i: (1, 0)),
            pl.BlockSpec((1, D_OUT), lambda i: (0, 0)),
        ],
        out_specs=pl.BlockSpec((R, D_OUT), lambda i: (i, 0)),
        out_shape=jax.ShapeDtypeStruct((N_NODES, D_OUT), jnp.float32),
    )(h2, h2, Wt, Wt, b.reshape(1, D_OUT))


def kernel(x, edge_index, W, b):
    src = edge_index[0].astype(jnp.int32)
    dst = edge_index[1].astype(jnp.int32)
    pad = E_PAD - N_EDGES
    x2 = x.reshape(N_NODES, 2, D_HALF).reshape(2 * N_NODES, D_HALF)
    g0 = jnp.concatenate([src * 2, jnp.zeros((pad,), jnp.int32)])
    dstp = jnp.concatenate([dst, jnp.full((pad,), N_NODES, jnp.int32)])
    base = dstp * 65536 + g0
    packed = jnp.stack([base, base + 1]).reshape(2, NS, N_CHUNKS // 4, 4 * CHUNK)
    h2 = _sc_segment_gather_sum(packed, x2)
    return _linear(h2, W.T, b)
